# baseline (speedup 1.0000x reference)
import jax
import jax.numpy as jnp
from jax import lax
from jax.experimental import pallas as pl
from jax.experimental.pallas import tpu as pltpu
from jax.experimental.pallas import tpu_sc as plsc

N_FZ, N_KEZ, N_ALPHA, N_TINT = 16, 8, 512, 50
N_ORB, N_TIMES = 1024, 200
T_PAD = 256
TINT_PAD = 64
NW = 32
MAX_TPW = 7
SLAB = N_ALPHA * N_TINT


def _prep_body(alphas_r, fzs_r, kezs_r, kezv_r, alpha_r, dmag_r, fzv_r,
               rows_r, fzrow_r, kez_r):
    la = jnp.log10(alphas_r[0, :])
    la0 = la[0]
    inv_da = 1.0 / (la[1] - la0)
    amin = alphas_r[0, 0]
    amax = alphas_r[0, N_ALPHA - 1]
    a = alpha_r[...]
    a_ind = (jnp.log10(a) - la0) * inv_da
    a0 = jnp.clip(a_ind.astype(jnp.int32), 0, N_ALPHA - 1)
    dal = a_ind - a0.astype(jnp.float32)
    idx0 = jnp.minimum(a0, N_ALPHA - 2) * N_TINT
    geom = (a >= amin) & (a <= amax)
    thr = jnp.where(geom, dmag_r[...], jnp.float32(jnp.inf))
    rows_r[:, 0:N_ORB] = lax.bitcast_convert_type(idx0, jnp.float32).T
    rows_r[:, N_ORB:2 * N_ORB] = dal.T
    rows_r[:, 2 * N_ORB:3 * N_ORB] = thr.T
    lf = jnp.log10(fzs_r[0, :])
    lf0 = lf[0]
    inv_df = 1.0 / (lf[1] - lf0)
    fz_ind = (jnp.log10(fzv_r[0, :]) - lf0) * inv_df
    fz0 = jnp.clip(jnp.floor(fz_ind).astype(jnp.int32) + 1, 0, N_FZ - 2)
    kez = jnp.sum((kezs_r[0, :] <= kezv_r[0, 0]).astype(jnp.int32)) - 1
    kez = jnp.clip(kez, 0, N_KEZ - 1)
    fzrow_r[...] = fz0[None, :]
    kez_r[...] = kez[None, None]


def _sc_body(grid_hbm, rows_hbm, fzrow_hbm, out_hbm,
             slab_a, slab_b, rows_a, rows_b, acc_v, outbuf_v, fzrow_v,
             sem_sa, sem_sb, sem_ra, sem_rb):
    c = lax.axis_index("c")
    s = lax.axis_index("s")
    wid = s * 2 + c
    pltpu.sync_copy(fzrow_hbm, fzrow_v)
    iota = lax.iota(jnp.int32, 16)
    zeros16 = jnp.zeros((16,), jnp.float32)
    inv_orb = jnp.float32(1.0 / N_ORB)
    G = 10
    t_tail = jnp.int32(192) + wid // 4
    o_tail = (wid % 4) * 16

    def start_fetch(t, slab_v, rows_v, sem_s, sem_r):
        row = fzrow_v[pl.ds(t, 16)][0]
        pltpu.async_copy(grid_hbm.at[row], slab_v, sem_s)
        pltpu.async_copy(rows_hbm.at[t], rows_v, sem_r)

    def wait_fetch(t, slab_v, rows_v, sem_s, sem_r):
        row = fzrow_v[pl.ds(t, 16)][0]
        pltpu.make_async_copy(grid_hbm.at[row], slab_v, sem_s).wait()
        pltpu.make_async_copy(rows_hbm.at[t], rows_v, sem_r).wait()

    def compute(i, slab_v, rows_v, o_lo, o_hi):
        def g_body(g, _):
            tbase = g * G

            def o_body(o, accs):
                ob = o * 16
                vbase = plsc.bitcast(rows_v[pl.ds(ob, 16)], jnp.int32)
                vdal = rows_v[pl.ds(N_ORB + ob, 16)]
                vthr = rows_v[pl.ds(2 * N_ORB + ob, 16)]
                vi = vbase + tbase
                new = []
                for k in range(G):
                    vik = vi + k
                    g0 = plsc.load_gather(slab_v, [vik])
                    g1 = plsc.load_gather(slab_v, [vik + N_TINT])
                    val = g0 + vdal * (g1 - g0)
                    det = vthr < val
                    new.append(accs[k] + jnp.where(det, 1.0, 0.0))
                return tuple(new)

            static = isinstance(o_lo, int) and isinstance(o_hi, int)
            accs = lax.fori_loop(o_lo, o_hi, o_body, (zeros16,) * G,
                                 unroll=2 if static else 1)
            for k in range(G):
                acc_v[pl.ds((tbase + k) * 16, 16)] = accs[k]
            return 0

        lax.fori_loop(0, N_TINT // G, g_body, 0)

        for j in range(TINT_PAD // 16):
            vrow = (iota + 16 * j) * 16
            ssum = zeros16
            for l in range(16):
                ssum = ssum + plsc.load_gather(acc_v, [vrow + l])
            outbuf_v[pl.ds(i * TINT_PAD + 16 * j, 16)] = ssum * inv_orb

    n_stage = MAX_TPW
    slabs = (slab_a, slab_b)
    rows = (rows_a, rows_b)
    sems_s = (sem_sa, sem_sb)
    sems_r = (sem_ra, sem_rb)

    def t_of(i):
        return t_tail if i == n_stage - 1 else wid + NW * i

    start_fetch(t_of(0), slabs[0], rows[0], sems_s[0], sems_r[0])
    for i in range(n_stage):
        b = i % 2
        wait_fetch(t_of(i), slabs[b], rows[b], sems_s[b], sems_r[b])
        if i + 1 < n_stage:
            nb = (i + 1) % 2
            start_fetch(t_of(i + 1), slabs[nb], rows[nb], sems_s[nb], sems_r[nb])
        if i == n_stage - 1:
            compute(i, slabs[b], rows[b], o_tail, o_tail + 16)
        else:
            compute(i, slabs[b], rows[b], 0, N_ORB // 16)

    pltpu.sync_copy(outbuf_v, out_hbm.at[wid])


def kernel(fZs, kEZs, alphas, int_times, grid, alpha, dMag, fZ_vals, kEZ_vals):
    del int_times
    f32, i32 = jnp.float32, jnp.int32
    alpha_p = jnp.pad(alpha, ((0, 0), (0, T_PAD - N_TIMES)), constant_values=0.1)
    dmag_p = jnp.pad(dMag, ((0, 0), (0, T_PAD - N_TIMES)), constant_values=0.0)
    fzv_p = jnp.pad(fZ_vals, (0, T_PAD - N_TIMES), constant_values=1.0)

    prep = pl.pallas_call(
        _prep_body,
        out_shape=[
            jax.ShapeDtypeStruct((T_PAD, 3 * N_ORB), f32),
            jax.ShapeDtypeStruct((1, T_PAD), i32),
            jax.ShapeDtypeStruct((1, 1), i32),
        ],
    )
    rows3, fzrow, kez_s = prep(
        alphas.reshape(1, N_ALPHA), fZs.reshape(1, N_FZ),
        kEZs.reshape(1, N_KEZ), kEZ_vals.reshape(1, 1),
        alpha_p, dmag_p, fzv_p.reshape(1, T_PAD))
    fzrow = fzrow.reshape(T_PAD)

    gslice = lax.dynamic_index_in_dim(grid, kez_s.reshape(()), axis=1,
                                      keepdims=False)
    grid2 = gslice.reshape(N_FZ, SLAB)
    mesh = plsc.VectorSubcoreMesh(core_axis_name="c", subcore_axis_name="s")
    sc = pl.kernel(
        _sc_body,
        out_type=jax.ShapeDtypeStruct((NW, MAX_TPW * TINT_PAD), f32),
        mesh=mesh,
        compiler_params=pltpu.CompilerParams(needs_layout_passes=False),
        scratch_types=[
            pltpu.VMEM((SLAB,), f32),
            pltpu.VMEM((SLAB,), f32),
            pltpu.VMEM((3 * N_ORB,), f32),
            pltpu.VMEM((3 * N_ORB,), f32),
            pltpu.VMEM((TINT_PAD * 16,), f32),
            pltpu.VMEM((MAX_TPW * TINT_PAD,), f32),
            pltpu.VMEM((T_PAD,), i32),
            pltpu.SemaphoreType.DMA,
            pltpu.SemaphoreType.DMA,
            pltpu.SemaphoreType.DMA,
            pltpu.SemaphoreType.DMA,
        ],
    )
    out = sc(grid2, rows3, fzrow)
    out3 = out.reshape(NW, MAX_TPW, TINT_PAD)
    full = out3[:, :MAX_TPW - 1].transpose(1, 0, 2).reshape((MAX_TPW - 1) * NW,
                                                            TINT_PAD)
    tail = out3[:, MAX_TPW - 1].reshape(8, 4, TINT_PAD).sum(axis=1)
    return jnp.concatenate([full, tail], axis=0)[:, :N_TINT]


# speedup vs baseline: 1.1528x; 1.1528x over previous
import jax
import jax.numpy as jnp
from jax import lax
from jax.experimental import pallas as pl
from jax.experimental.pallas import tpu as pltpu
from jax.experimental.pallas import tpu_sc as plsc

N_FZ, N_KEZ, N_ALPHA, N_TINT = 16, 8, 512, 50
N_ORB, N_TIMES = 1024, 200
T_PAD = 256
TINT_PAD = 64
NW = 32
MAX_TPW = 7
SLAB = N_ALPHA * N_TINT


def _prep_body(alphas_r, fzs_r, kezs_r, kezv_r, alpha_r, dmag_r, fzv_r,
               rows_r, fzrow_r, kez_r):
    la = jnp.log10(alphas_r[0, :])
    la0 = la[0]
    inv_da = 1.0 / (la[1] - la0)
    amin = alphas_r[0, 0]
    amax = alphas_r[0, N_ALPHA - 1]
    a = alpha_r[...]
    a_ind = (jnp.log10(a) - la0) * inv_da
    a0 = jnp.clip(a_ind.astype(jnp.int32), 0, N_ALPHA - 1)
    dal = a_ind - a0.astype(jnp.float32)
    idx0 = jnp.minimum(a0, N_ALPHA - 2) * N_TINT
    geom = (a >= amin) & (a <= amax)
    thr = jnp.where(geom, dmag_r[...], jnp.float32(jnp.inf))
    rows_r[:, 0:N_ORB] = lax.bitcast_convert_type(idx0, jnp.float32).T
    rows_r[:, N_ORB:2 * N_ORB] = dal.T
    rows_r[:, 2 * N_ORB:3 * N_ORB] = thr.T
    lf = jnp.log10(fzs_r[0, :])
    lf0 = lf[0]
    inv_df = 1.0 / (lf[1] - lf0)
    fz_ind = (jnp.log10(fzv_r[0, :]) - lf0) * inv_df
    fz0 = jnp.clip(jnp.floor(fz_ind).astype(jnp.int32) + 1, 0, N_FZ - 2)
    kez = jnp.sum((kezs_r[0, :] <= kezv_r[0, 0]).astype(jnp.int32)) - 1
    kez = jnp.clip(kez, 0, N_KEZ - 1)
    fzrow_r[...] = fz0[None, :]
    kez_r[...] = kez[None, None]


def _sc_body(grid_hbm, rows_hbm, fzrow_hbm, out_hbm,
             slab_a, slab_b, rows_a, rows_b, acc_v, outbuf_v, fzrow_v,
             sem_sa, sem_sb, sem_ra, sem_rb):
    c = lax.axis_index("c")
    s = lax.axis_index("s")
    wid = s * 2 + c
    pltpu.sync_copy(fzrow_hbm, fzrow_v)
    iota = lax.iota(jnp.int32, 16)
    zeros16 = jnp.zeros((16,), jnp.float32)
    inv_orb = jnp.float32(1.0 / N_ORB)
    G = 10
    t_tail = jnp.int32(192) + wid // 4
    o_tail = (wid % 4) * 16

    def start_fetch(t, slab_v, rows_v, sem_s, sem_r):
        row = fzrow_v[pl.ds(t, 16)][0]
        pltpu.async_copy(grid_hbm.at[row], slab_v, sem_s)
        pltpu.async_copy(rows_hbm.at[t], rows_v, sem_r)

    def wait_fetch(t, slab_v, rows_v, sem_s, sem_r):
        row = fzrow_v[pl.ds(t, 16)][0]
        pltpu.make_async_copy(grid_hbm.at[row], slab_v, sem_s).wait()
        pltpu.make_async_copy(rows_hbm.at[t], rows_v, sem_r).wait()

    def compute(i, slab_v, rows_v, o_lo, o_hi):
        def g_body(g, _):
            tbase = g * G

            def o_body(o, accs):
                ob = o * 16
                vbase = plsc.bitcast(rows_v[pl.ds(ob, 16)], jnp.int32)
                vdal = rows_v[pl.ds(N_ORB + ob, 16)]
                vthr = rows_v[pl.ds(2 * N_ORB + ob, 16)]
                vi = vbase + tbase
                new = []
                for k in range(G):
                    vik = vi + k
                    g0 = plsc.load_gather(slab_v, [vik])
                    g1 = plsc.load_gather(slab_v, [vik + N_TINT])
                    val = g0 + vdal * (g1 - g0)
                    det = vthr < val
                    new.append(accs[k] + jnp.where(det, 1.0, 0.0))
                return tuple(new)

            accs = lax.fori_loop(o_lo, o_hi, o_body, (zeros16,) * G)
            for k in range(G):
                acc_v[pl.ds((tbase + k) * 16, 16)] = accs[k]
            return 0

        lax.fori_loop(0, N_TINT // G, g_body, 0)

        for j in range(TINT_PAD // 16):
            vrow = (iota + 16 * j) * 16
            ssum = zeros16
            for l in range(16):
                ssum = ssum + plsc.load_gather(acc_v, [vrow + l])
            outbuf_v[pl.ds(i * TINT_PAD + 16 * j, 16)] = ssum * inv_orb

    n_stage = MAX_TPW
    slabs = (slab_a, slab_b)
    rows = (rows_a, rows_b)
    sems_s = (sem_sa, sem_sb)
    sems_r = (sem_ra, sem_rb)

    def t_of(i):
        return t_tail if i == n_stage - 1 else wid + NW * i

    start_fetch(t_of(0), slabs[0], rows[0], sems_s[0], sems_r[0])
    for i in range(n_stage):
        b = i % 2
        wait_fetch(t_of(i), slabs[b], rows[b], sems_s[b], sems_r[b])
        if i + 1 < n_stage:
            nb = (i + 1) % 2
            start_fetch(t_of(i + 1), slabs[nb], rows[nb], sems_s[nb], sems_r[nb])
        if i == n_stage - 1:
            compute(i, slabs[b], rows[b], o_tail, o_tail + 16)
        else:
            compute(i, slabs[b], rows[b], 0, N_ORB // 16)

    pltpu.sync_copy(outbuf_v, out_hbm.at[wid])


def kernel(fZs, kEZs, alphas, int_times, grid, alpha, dMag, fZ_vals, kEZ_vals):
    del int_times
    f32, i32 = jnp.float32, jnp.int32
    alpha_p = jnp.pad(alpha, ((0, 0), (0, T_PAD - N_TIMES)), constant_values=0.1)
    dmag_p = jnp.pad(dMag, ((0, 0), (0, T_PAD - N_TIMES)), constant_values=0.0)
    fzv_p = jnp.pad(fZ_vals, (0, T_PAD - N_TIMES), constant_values=1.0)

    prep = pl.pallas_call(
        _prep_body,
        out_shape=[
            jax.ShapeDtypeStruct((T_PAD, 3 * N_ORB), f32),
            jax.ShapeDtypeStruct((1, T_PAD), i32),
            jax.ShapeDtypeStruct((1, 1), i32),
        ],
    )
    rows3, fzrow, kez_s = prep(
        alphas.reshape(1, N_ALPHA), fZs.reshape(1, N_FZ),
        kEZs.reshape(1, N_KEZ), kEZ_vals.reshape(1, 1),
        alpha_p, dmag_p, fzv_p.reshape(1, T_PAD))
    fzrow = fzrow.reshape(T_PAD)

    gslice = lax.dynamic_index_in_dim(grid, kez_s.reshape(()), axis=1,
                                      keepdims=False)
    grid2 = gslice.reshape(N_FZ, SLAB)
    mesh = plsc.VectorSubcoreMesh(core_axis_name="c", subcore_axis_name="s")
    sc = pl.kernel(
        _sc_body,
        out_type=jax.ShapeDtypeStruct((NW, MAX_TPW * TINT_PAD), f32),
        mesh=mesh,
        compiler_params=pltpu.CompilerParams(needs_layout_passes=False),
        scratch_types=[
            pltpu.VMEM((SLAB,), f32),
            pltpu.VMEM((SLAB,), f32),
            pltpu.VMEM((3 * N_ORB,), f32),
            pltpu.VMEM((3 * N_ORB,), f32),
            pltpu.VMEM((TINT_PAD * 16,), f32),
            pltpu.VMEM((MAX_TPW * TINT_PAD,), f32),
            pltpu.VMEM((T_PAD,), i32),
            pltpu.SemaphoreType.DMA,
            pltpu.SemaphoreType.DMA,
            pltpu.SemaphoreType.DMA,
            pltpu.SemaphoreType.DMA,
        ],
    )
    out = sc(grid2, rows3, fzrow)
    out3 = out.reshape(NW, MAX_TPW, TINT_PAD)
    full = out3[:, :MAX_TPW - 1].transpose(1, 0, 2).reshape((MAX_TPW - 1) * NW,
                                                            TINT_PAD)
    tail = out3[:, MAX_TPW - 1].reshape(8, 4, TINT_PAD).sum(axis=1)
    return jnp.concatenate([full, tail], axis=0)[:, :N_TINT]
